# baseline (speedup 1.0000x reference)
import jax
import jax.numpy as jnp
from jax.experimental import pallas as pl
from jax.experimental.pallas import tpu as pltpu

_PS = 128


def _make_body(B):
    def _add_body(in_ref, pe_ref, out_ref):
        pe = pe_ref[...]
        pe_rep = jnp.repeat(pe, B, axis=0)
        out_ref[...] = in_ref[...] + pe_rep

    return _add_body


def kernel(input, pe_table):
    S, B, D = input.shape
    x = input.reshape(S * B, D)
    grid = (S // _PS,)
    out = pl.pallas_call(
        _make_body(B),
        grid=grid,
        in_specs=[
            pl.BlockSpec((_PS * B, D), lambda i: (i, 0)),
            pl.BlockSpec((_PS, D), lambda i: (i, 0)),
        ],
        out_specs=pl.BlockSpec((_PS * B, D), lambda i: (i, 0)),
        out_shape=jax.ShapeDtypeStruct((S * B, D), input.dtype),
        compiler_params=pltpu.CompilerParams(
            dimension_semantics=("arbitrary",),
        ),
    )(x, pe_table)
    return out.reshape(S, B, D)


# speedup vs baseline: 4.1110x; 4.1110x over previous
import jax
import jax.numpy as jnp
from jax.experimental import pallas as pl
from jax.experimental.pallas import tpu as pltpu

_BS = 512


def _add_body(in_ref, pe_ref, out_ref):
    out_ref[...] = in_ref[...] + pe_ref[...][:, None, :]


def kernel(input, pe_table):
    S, B, D = input.shape
    grid = (S // _BS,)
    return pl.pallas_call(
        _add_body,
        grid=grid,
        in_specs=[
            pl.BlockSpec((_BS, B, D), lambda i: (i, 0, 0)),
            pl.BlockSpec((_BS, D), lambda i: (i, 0)),
        ],
        out_specs=pl.BlockSpec((_BS, B, D), lambda i: (i, 0, 0)),
        out_shape=jax.ShapeDtypeStruct((S, B, D), input.dtype),
        compiler_params=pltpu.CompilerParams(
            dimension_semantics=("arbitrary",),
        ),
    )(input, pe_table)
